# baseline (speedup 1.0000x reference)
import functools
import math

import jax
import jax.numpy as jnp
import numpy as np
from jax import lax
from jax.experimental import pallas as pl

_H = (8, 16)
_K = (32, 32)
_PF = (0.25, 0.25)
_BF = jnp.bfloat16
_F32 = jnp.float32
_HI = lax.Precision.HIGHEST


def _bdot(a, b):
    return jnp.dot(a.astype(_BF), b.astype(_BF), preferred_element_type=_F32)


def _qkv_knl(x_ref, wq, bq, wk, bk, wv, bv, q_ref, k_ref, v_ref):
    x = x_ref[...]
    q_ref[...] = _bdot(x, wq[...]) + bq[...]
    k_ref[...] = _bdot(x, wk[...]) + bk[...]
    v_ref[...] = _bdot(x, wv[...]) + bv[...]


def _res_knl(x_ref, o_ref, w, b, out_ref):
    out_ref[...] = x_ref[...] + _bdot(o_ref[...], w[...]) + b[...]


def _ff_knl(x_ref, w1, b1, w2, b2, p1, pb1, xm_ref, h_ref):
    x = x_ref[...]
    ff = jnp.maximum(_bdot(x, w1[...]) + b1[...], 0.0)
    xm = x + _bdot(ff, w2[...]) + b2[...]
    xm_ref[...] = xm
    h_ref[...] = jnp.tanh(_bdot(xm, p1[...]) + pb1[...])


def _emb_knl(x_ref, w, b, out_ref):
    out_ref[...] = _bdot(x_ref[...], w[...]) + b[...]


def _pool_knl(x_ref, sc_ref, sr_ref, xv_ref, oW_ref, ob_ref,
              xo_ref, xvo_ref, vals_ref, idx_ref, *, N, M, RC):
    s_col = sc_ref[0]
    s_row = sr_ref[0]
    parts = []
    for c in range(N // RC):
        sn = s_row[:, c * RC:(c + 1) * RC]
        gt = (s_col > sn).astype(_F32)
        eq = s_col == sn
        jj = lax.broadcasted_iota(jnp.int32, (N, RC), 0)
        nn = lax.broadcasted_iota(jnp.int32, (N, RC), 1) + c * RC
        tie = jnp.where(eq, (jj < nn).astype(_F32), 0.0)
        parts.append(jnp.sum(gt + tie, axis=0, keepdims=True))
    rank = jnp.concatenate(parts, axis=1)
    om = lax.broadcasted_iota(jnp.int32, (M, 1), 0).astype(_F32)
    P = (rank == om).astype(_F32)
    x_pool = jnp.dot(P, x_ref[0], preferred_element_type=_F32, precision=_HI)
    xv_pool = jnp.dot(P, xv_ref[0], preferred_element_type=_F32, precision=_HI)
    vals = jnp.dot(P, s_col, preferred_element_type=_F32, precision=_HI)
    ii = lax.broadcasted_iota(jnp.int32, (N, 1), 0).astype(_F32)
    idxf = jnp.dot(P, ii, preferred_element_type=_F32, precision=_HI)
    gate = jax.nn.sigmoid(vals)
    xo_ref[0] = _bdot(x_pool * gate, oW_ref[...]) + ob_ref[...]
    xvo_ref[0] = xv_pool
    vals_ref[0] = vals
    idx_ref[0] = jnp.round(idxf).astype(jnp.int32)


def _lin_call(knl, args, out_shapes):
    return pl.pallas_call(knl, out_shape=out_shapes)(*args)


def _row(a):
    return a.reshape(1, -1)


def _knn_idx(xv, K):
    n2 = jnp.sum(xv * xv, axis=-1)
    d2 = n2[:, :, None] + n2[:, None, :] - 2.0 * jnp.einsum('bnd,bmd->bnm', xv, xv)
    _, idx = jax.lax.top_k(-d2, K)
    return idx


def _gather(a, idx):
    return jax.vmap(lambda ab, ib: ab[ib])(a, idx)


def _batch_spec(shape):
    nd = len(shape)
    return pl.BlockSpec((1,) + shape[1:], lambda b: (b,) + (0,) * (nd - 1))


def _full_spec(shape):
    nd = len(shape)
    return pl.BlockSpec(shape, lambda b: (0,) * nd)


def _run_pool(x_mid, s, xv_pad, oW, ob, *, M, OUT):
    B, N, D = x_mid.shape
    s_col = s[..., None]
    s_row = s[:, None, :]
    body = functools.partial(_pool_knl, N=N, M=M, RC=512)
    in_specs = [_batch_spec(x_mid.shape), _batch_spec(s_col.shape),
                _batch_spec(s_row.shape), _batch_spec(xv_pad.shape),
                _full_spec(oW.shape), _full_spec(ob.shape)]
    out_shape = (jax.ShapeDtypeStruct((B, M, OUT), _F32),
                 jax.ShapeDtypeStruct((B, M, 8), _F32),
                 jax.ShapeDtypeStruct((B, M, 1), _F32),
                 jax.ShapeDtypeStruct((B, M, 1), jnp.int32))
    out_specs = (_batch_spec((B, M, OUT)), _batch_spec((B, M, 8)),
                 _batch_spec((B, M, 1)), _batch_spec((B, M, 1)))
    fn = pl.pallas_call(body, grid=(B,), in_specs=in_specs,
                        out_specs=out_specs, out_shape=out_shape)
    return fn(x_mid, s_col, s_row, xv_pad, oW, ob)


def kernel(x, x_v, params):
    p = params
    B, N, _ = x.shape
    x2 = x.reshape(B * N, 3)
    xe = _lin_call(_emb_knl, (x2, p['emb_W'], _row(p['emb_b'])),
                   jax.ShapeDtypeStruct((B * N, 128), _F32))
    x = xe.reshape(B, N, 128)
    extras = []
    for i in range(2):
        H, K = _H[i], _K[i]
        B, N, D = x.shape
        dh = D // H
        idx = _knn_idx(x_v, K)
        x2 = x.reshape(B * N, D)
        sd = jax.ShapeDtypeStruct((B * N, D), _F32)
        q, k, v = _lin_call(
            _qkv_knl,
            (x2, p[f'mha{i}_Wq'], _row(p[f'mha{i}_Wqb']),
             p[f'mha{i}_Wk'], _row(p[f'mha{i}_Wkb']),
             p[f'mha{i}_Wv'], _row(p[f'mha{i}_Wvb'])),
            (sd, sd, sd))
        q = q.reshape(B, N, D)
        k = k.reshape(B, N, D)
        v = v.reshape(B, N, D)
        kn = _gather(k, idx).reshape(B, N, K, H, dh)
        vn = _gather(v, idx).reshape(B, N, K, H, dh)
        qh = q.reshape(B, N, H, dh)
        logits = jnp.einsum('bnhd,bnkhd->bnhk', qh, kn) / np.sqrt(dh)
        w = jax.nn.softmax(logits, axis=-1)
        o = jnp.einsum('bnhk,bnkhd->bnhd', w, vn).reshape(B * N, D)
        xa = _lin_call(_res_knl,
                       (x2, o, p[f'mha{i}_Wo'], _row(p[f'mha{i}_Wob'])), sd)
        FFD = p[f'ff{i}_W1'].shape[1]
        PH = p[f'pool{i}_W1'].shape[1]
        xm, h = _lin_call(
            _ff_knl,
            (xa, p[f'ff{i}_W1'], _row(p[f'ff{i}_b1']),
             p[f'ff{i}_W2'], _row(p[f'ff{i}_b2']),
             p[f'pool{i}_W1'], _row(p[f'pool{i}_b1'])),
            (sd, jax.ShapeDtypeStruct((B * N, PH), _F32)))
        s = (h @ p[f'pool{i}_W2'] + p[f'pool{i}_b2'])[..., 0].reshape(B, N)
        M = int(N * _PF[i])
        OUT = p[f'out{i}_W'].shape[1]
        xv_pad = jnp.pad(x_v, ((0, 0), (0, 0), (0, 5)))
        xn, xvn, vals, idxs = _run_pool(
            xm.reshape(B, N, D), s, xv_pad, p[f'out{i}_W'],
            _row(p[f'out{i}_b']), M=M, OUT=OUT)
        x = xn
        x_v = xvn[..., :3]
        extras.append((vals[..., 0], idxs[..., 0]))
    return (x, x_v, extras[0][0], extras[0][1], extras[1][0], extras[1][1])


# speedup vs baseline: 1.8714x; 1.8714x over previous
import functools
import math

import jax
import jax.numpy as jnp
import numpy as np
from jax import lax
from jax.experimental import pallas as pl

_H = (8, 16)
_K = (32, 32)
_PF = (0.25, 0.25)
_BF = jnp.bfloat16
_F32 = jnp.float32
_HI = lax.Precision.HIGHEST


def _bdot(a, b):
    return jnp.dot(a.astype(_BF), b.astype(_BF), preferred_element_type=_F32)


def _qkv_knl(x_ref, wq, bq, wk, bk, wv, bv, q_ref, k_ref, v_ref):
    x = x_ref[...]
    q_ref[...] = _bdot(x, wq[...]) + bq[...]
    k_ref[...] = _bdot(x, wk[...]) + bk[...]
    v_ref[...] = _bdot(x, wv[...]) + bv[...]


def _res_knl(x_ref, o_ref, w, b, out_ref):
    out_ref[...] = x_ref[...] + _bdot(o_ref[...], w[...]) + b[...]


def _ff_knl(x_ref, w1, b1, w2, b2, p1, pb1, xm_ref, h_ref):
    x = x_ref[...]
    ff = jnp.maximum(_bdot(x, w1[...]) + b1[...], 0.0)
    xm = x + _bdot(ff, w2[...]) + b2[...]
    xm_ref[...] = xm
    h_ref[...] = jnp.tanh(_bdot(xm, p1[...]) + pb1[...])


def _emb_knl(x_ref, w, b, out_ref):
    out_ref[...] = _bdot(x_ref[...], w[...]) + b[...]


def _pool_knl(x_ref, sc_ref, sr_ref, xv_ref, oW_ref, ob_ref,
              xo_ref, xvo_ref, vals_ref, idx_ref, *, N, M, RC):
    s_col = sc_ref[0]
    s_row = sr_ref[0]
    parts = []
    for c in range(N // RC):
        sn = s_row[:, c * RC:(c + 1) * RC]
        gt = (s_col > sn).astype(_F32)
        eq = s_col == sn
        jj = lax.broadcasted_iota(jnp.int32, (N, RC), 0)
        nn = lax.broadcasted_iota(jnp.int32, (N, RC), 1) + c * RC
        tie = jnp.where(eq, (jj < nn).astype(_F32), 0.0)
        parts.append(jnp.sum(gt + tie, axis=0, keepdims=True))
    rank = jnp.concatenate(parts, axis=1)
    om = lax.broadcasted_iota(jnp.int32, (M, 1), 0).astype(_F32)
    P = (rank == om).astype(_F32)
    x_pool = jnp.dot(P, x_ref[0], preferred_element_type=_F32, precision=_HI)
    xv_pool = jnp.dot(P, xv_ref[0], preferred_element_type=_F32, precision=_HI)
    vals = jnp.dot(P, s_col, preferred_element_type=_F32, precision=_HI)
    ii = lax.broadcasted_iota(jnp.int32, (N, 1), 0).astype(_F32)
    idxf = jnp.dot(P, ii, preferred_element_type=_F32, precision=_HI)
    gate = jax.nn.sigmoid(vals)
    xo_ref[0] = _bdot(x_pool * gate, oW_ref[...]) + ob_ref[...]
    xvo_ref[0] = xv_pool
    vals_ref[0] = vals
    idx_ref[0] = jnp.round(idxf).astype(jnp.int32)


def _gat_knl(idx_ref, kp0, kp1, kp2, vp0, vp1, vp2, kn_ref, vn_ref, *, CH, N):
    idx_col = idx_ref[0]
    iota = lax.broadcasted_iota(jnp.int32, (CH, N), 1)
    G = (idx_col == iota).astype(_BF)
    def g3(p0, p1, p2):
        a = jnp.dot(G, p0[0], preferred_element_type=_F32)
        b = jnp.dot(G, p1[0], preferred_element_type=_F32)
        c = jnp.dot(G, p2[0], preferred_element_type=_F32)
        return (a + b) + c
    kn_ref[0] = g3(kp0, kp1, kp2)
    vn_ref[0] = g3(vp0, vp1, vp2)


def _bf3_split(a):
    bits = lax.bitcast_convert_type(a, jnp.uint32)
    p0 = lax.bitcast_convert_type(bits & jnp.uint32(0xFFFF0000), jnp.float32)
    r1 = a - p0
    bits1 = lax.bitcast_convert_type(r1, jnp.uint32)
    p1 = lax.bitcast_convert_type(bits1 & jnp.uint32(0xFFFF0000), jnp.float32)
    p2 = r1 - p1
    return p0.astype(_BF), p1.astype(_BF), p2.astype(_BF)


def _gather_kv(k, v, idx):
    B, N, D = k.shape
    K = idx.shape[-1]
    NK = N * K
    CH = 1024 if NK % 1024 == 0 else 512
    NC = NK // CH
    idx3 = idx.reshape(B * NC, CH, 1)
    ks = [p.reshape(B, N, D) for p in _bf3_split(k.reshape(B, N, D))]
    vs = [p.reshape(B, N, D) for p in _bf3_split(v.reshape(B, N, D))]
    body = functools.partial(_gat_knl, CH=CH, N=N)
    in_specs = [pl.BlockSpec((1, CH, 1), lambda b, c: (b * NC + c, 0, 0))]
    in_specs += [pl.BlockSpec((1, N, D), lambda b, c: (b, 0, 0))] * 6
    out_specs = (pl.BlockSpec((1, CH, D), lambda b, c: (b, c, 0)),) * 2
    out_shape = (jax.ShapeDtypeStruct((B, NK, D), _F32),) * 2
    fn = pl.pallas_call(body, grid=(B, NC), in_specs=in_specs,
                        out_specs=out_specs, out_shape=out_shape)
    NCc = NC
    kn, vn = fn(idx3, *ks, *vs)
    return kn, vn


def _lin_call(knl, args, out_shapes):
    return pl.pallas_call(knl, out_shape=out_shapes)(*args)


def _row(a):
    return a.reshape(1, -1)


def _knn_idx(xv, K):
    n2 = jnp.sum(xv * xv, axis=-1)
    d2 = n2[:, :, None] + n2[:, None, :] - 2.0 * jnp.einsum('bnd,bmd->bnm', xv, xv)
    _, idx = jax.lax.top_k(-d2, K)
    return idx


def _gather(a, idx):
    return jax.vmap(lambda ab, ib: ab[ib])(a, idx)


def _batch_spec(shape):
    nd = len(shape)
    return pl.BlockSpec((1,) + shape[1:], lambda b: (b,) + (0,) * (nd - 1))


def _full_spec(shape):
    nd = len(shape)
    return pl.BlockSpec(shape, lambda b: (0,) * nd)


def _run_pool(x_mid, s, xv_pad, oW, ob, *, M, OUT):
    B, N, D = x_mid.shape
    s_col = s[..., None]
    s_row = s[:, None, :]
    body = functools.partial(_pool_knl, N=N, M=M, RC=512)
    in_specs = [_batch_spec(x_mid.shape), _batch_spec(s_col.shape),
                _batch_spec(s_row.shape), _batch_spec(xv_pad.shape),
                _full_spec(oW.shape), _full_spec(ob.shape)]
    out_shape = (jax.ShapeDtypeStruct((B, M, OUT), _F32),
                 jax.ShapeDtypeStruct((B, M, 8), _F32),
                 jax.ShapeDtypeStruct((B, M, 1), _F32),
                 jax.ShapeDtypeStruct((B, M, 1), jnp.int32))
    out_specs = (_batch_spec((B, M, OUT)), _batch_spec((B, M, 8)),
                 _batch_spec((B, M, 1)), _batch_spec((B, M, 1)))
    fn = pl.pallas_call(body, grid=(B,), in_specs=in_specs,
                        out_specs=out_specs, out_shape=out_shape)
    return fn(x_mid, s_col, s_row, xv_pad, oW, ob)


def kernel(x, x_v, params):
    p = params
    B, N, _ = x.shape
    x2 = x.reshape(B * N, 3)
    xe = _lin_call(_emb_knl, (x2, p['emb_W'], _row(p['emb_b'])),
                   jax.ShapeDtypeStruct((B * N, 128), _F32))
    x = xe.reshape(B, N, 128)
    extras = []
    for i in range(2):
        H, K = _H[i], _K[i]
        B, N, D = x.shape
        dh = D // H
        idx = _knn_idx(x_v, K)
        x2 = x.reshape(B * N, D)
        sd = jax.ShapeDtypeStruct((B * N, D), _F32)
        q, k, v = _lin_call(
            _qkv_knl,
            (x2, p[f'mha{i}_Wq'], _row(p[f'mha{i}_Wqb']),
             p[f'mha{i}_Wk'], _row(p[f'mha{i}_Wkb']),
             p[f'mha{i}_Wv'], _row(p[f'mha{i}_Wvb'])),
            (sd, sd, sd))
        q = q.reshape(B, N, D)
        k = k.reshape(B, N, D)
        v = v.reshape(B, N, D)
        kn2, vn2 = _gather_kv(k, v, idx)
        kn = kn2.reshape(B, N, K, H, dh)
        vn = vn2.reshape(B, N, K, H, dh)
        qh = q.reshape(B, N, H, dh)
        logits = jnp.einsum('bnhd,bnkhd->bnhk', qh, kn) / np.sqrt(dh)
        w = jax.nn.softmax(logits, axis=-1)
        o = jnp.einsum('bnhk,bnkhd->bnhd', w, vn).reshape(B * N, D)
        xa = _lin_call(_res_knl,
                       (x2, o, p[f'mha{i}_Wo'], _row(p[f'mha{i}_Wob'])), sd)
        FFD = p[f'ff{i}_W1'].shape[1]
        PH = p[f'pool{i}_W1'].shape[1]
        xm, h = _lin_call(
            _ff_knl,
            (xa, p[f'ff{i}_W1'], _row(p[f'ff{i}_b1']),
             p[f'ff{i}_W2'], _row(p[f'ff{i}_b2']),
             p[f'pool{i}_W1'], _row(p[f'pool{i}_b1'])),
            (sd, jax.ShapeDtypeStruct((B * N, PH), _F32)))
        s = (h @ p[f'pool{i}_W2'] + p[f'pool{i}_b2'])[..., 0].reshape(B, N)
        M = int(N * _PF[i])
        OUT = p[f'out{i}_W'].shape[1]
        xv_pad = jnp.pad(x_v, ((0, 0), (0, 0), (0, 5)))
        xn, xvn, vals, idxs = _run_pool(
            xm.reshape(B, N, D), s, xv_pad, p[f'out{i}_W'],
            _row(p[f'out{i}_b']), M=M, OUT=OUT)
        x = xn
        x_v = xvn[..., :3]
        extras.append((vals[..., 0], idxs[..., 0]))
    return (x, x_v, extras[0][0], extras[0][1], extras[1][0], extras[1][1])


# speedup vs baseline: 2.0868x; 1.1151x over previous
import functools
import math

import jax
import jax.numpy as jnp
import numpy as np
from jax import lax
from jax.experimental import pallas as pl

_H = (8, 16)
_K = (32, 32)
_PF = (0.25, 0.25)
_BF = jnp.bfloat16
_F32 = jnp.float32
_HI = lax.Precision.HIGHEST


def _bdot(a, b):
    return jnp.dot(a.astype(_BF), b.astype(_BF), preferred_element_type=_F32)


def _qkv_knl(x_ref, wq, bq, wk, bk, wv, bv, q_ref, k_ref, v_ref):
    x = x_ref[...]
    q_ref[...] = _bdot(x, wq[...]) + bq[...]
    k_ref[...] = _bdot(x, wk[...]) + bk[...]
    v_ref[...] = _bdot(x, wv[...]) + bv[...]


def _res_knl(x_ref, o_ref, w, b, out_ref):
    out_ref[...] = x_ref[...] + _bdot(o_ref[...], w[...]) + b[...]


def _ff_knl(x_ref, w1, b1, w2, b2, p1, pb1, xm_ref, h_ref):
    x = x_ref[...]
    ff = jnp.maximum(_bdot(x, w1[...]) + b1[...], 0.0)
    xm = x + _bdot(ff, w2[...]) + b2[...]
    xm_ref[...] = xm
    h_ref[...] = jnp.tanh(_bdot(xm, p1[...]) + pb1[...])


def _emb_knl(x_ref, w, b, out_ref):
    out_ref[...] = _bdot(x_ref[...], w[...]) + b[...]


def _pool_knl(x_ref, sc_ref, sr_ref, xv_ref, oW_ref, ob_ref,
              xo_ref, xvo_ref, vals_ref, idx_ref, *, N, M, RC):
    s_col = sc_ref[0]
    s_row = sr_ref[0]
    parts = []
    for c in range(N // RC):
        sn = s_row[:, c * RC:(c + 1) * RC]
        gt = (s_col > sn).astype(_F32)
        eq = s_col == sn
        jj = lax.broadcasted_iota(jnp.int32, (N, RC), 0)
        nn = lax.broadcasted_iota(jnp.int32, (N, RC), 1) + c * RC
        tie = jnp.where(eq, (jj < nn).astype(_F32), 0.0)
        parts.append(jnp.sum(gt + tie, axis=0, keepdims=True))
    rank = jnp.concatenate(parts, axis=1)
    om = lax.broadcasted_iota(jnp.int32, (M, 1), 0).astype(_F32)
    P = (rank == om).astype(_F32)
    x_pool = jnp.dot(P, x_ref[0], preferred_element_type=_F32, precision=_HI)
    xv_pool = jnp.dot(P, xv_ref[0], preferred_element_type=_F32, precision=_HI)
    vals = jnp.dot(P, s_col, preferred_element_type=_F32, precision=_HI)
    ii = lax.broadcasted_iota(jnp.int32, (N, 1), 0).astype(_F32)
    idxf = jnp.dot(P, ii, preferred_element_type=_F32, precision=_HI)
    gate = jax.nn.sigmoid(vals)
    xo_ref[0] = _bdot(x_pool * gate, oW_ref[...]) + ob_ref[...]
    xvo_ref[0] = xv_pool
    vals_ref[0] = vals
    idx_ref[0] = jnp.round(idxf).astype(jnp.int32)


def _gat_knl(idx_ref, p0_ref, p1_ref, p2_ref, kn_ref, vn_ref, *, CH, N, D):
    idx_col = idx_ref[0]
    iota = lax.broadcasted_iota(jnp.int32, (CH, N), 1)
    G = (idx_col == iota).astype(_BF)
    a = jnp.dot(G, p0_ref[0], preferred_element_type=_F32)
    b = jnp.dot(G, p1_ref[0], preferred_element_type=_F32)
    c = jnp.dot(G, p2_ref[0], preferred_element_type=_F32)
    kv = (a + b) + c
    kn_ref[0] = kv[:, :D]
    vn_ref[0] = kv[:, D:]


def _bf3_split(a):
    bits = lax.bitcast_convert_type(a, jnp.uint32)
    p0 = lax.bitcast_convert_type(bits & jnp.uint32(0xFFFF0000), jnp.float32)
    r1 = a - p0
    bits1 = lax.bitcast_convert_type(r1, jnp.uint32)
    p1 = lax.bitcast_convert_type(bits1 & jnp.uint32(0xFFFF0000), jnp.float32)
    p2 = r1 - p1
    return p0.astype(_BF), p1.astype(_BF), p2.astype(_BF)


def _gather_kv(k, v, idx):
    B, N, D = k.shape
    K = idx.shape[-1]
    NK = N * K
    CH = 2048 if NK % 2048 == 0 else 512
    NC = NK // CH
    idx3 = idx.reshape(B * NC, CH, 1)
    kv = jnp.concatenate([k, v], axis=-1)
    ps = [p for p in _bf3_split(kv)]
    body = functools.partial(_gat_knl, CH=CH, N=N, D=D)
    in_specs = [pl.BlockSpec((1, CH, 1), lambda b, c: (b * NC + c, 0, 0))]
    in_specs += [pl.BlockSpec((1, N, 2 * D), lambda b, c: (b, 0, 0))] * 3
    out_specs = (pl.BlockSpec((1, CH, D), lambda b, c: (b, c, 0)),) * 2
    out_shape = (jax.ShapeDtypeStruct((B, NK, D), _F32),) * 2
    fn = pl.pallas_call(body, grid=(B, NC), in_specs=in_specs,
                        out_specs=out_specs, out_shape=out_shape)
    kn, vn = fn(idx3, *ps)
    return kn, vn


def _lin_call(knl, args, out_shapes):
    return pl.pallas_call(knl, out_shape=out_shapes)(*args)


def _row(a):
    return a.reshape(1, -1)


def _knn_idx(xv, K):
    n2 = jnp.sum(xv * xv, axis=-1)
    d2 = n2[:, :, None] + n2[:, None, :] - 2.0 * jnp.einsum('bnd,bmd->bnm', xv, xv)
    _, idx = jax.lax.top_k(-d2, K)
    return idx


def _gather(a, idx):
    return jax.vmap(lambda ab, ib: ab[ib])(a, idx)


def _batch_spec(shape):
    nd = len(shape)
    return pl.BlockSpec((1,) + shape[1:], lambda b: (b,) + (0,) * (nd - 1))


def _full_spec(shape):
    nd = len(shape)
    return pl.BlockSpec(shape, lambda b: (0,) * nd)


def _run_pool(x_mid, s, xv_pad, oW, ob, *, M, OUT):
    B, N, D = x_mid.shape
    s_col = s[..., None]
    s_row = s[:, None, :]
    body = functools.partial(_pool_knl, N=N, M=M, RC=512)
    in_specs = [_batch_spec(x_mid.shape), _batch_spec(s_col.shape),
                _batch_spec(s_row.shape), _batch_spec(xv_pad.shape),
                _full_spec(oW.shape), _full_spec(ob.shape)]
    out_shape = (jax.ShapeDtypeStruct((B, M, OUT), _F32),
                 jax.ShapeDtypeStruct((B, M, 8), _F32),
                 jax.ShapeDtypeStruct((B, M, 1), _F32),
                 jax.ShapeDtypeStruct((B, M, 1), jnp.int32))
    out_specs = (_batch_spec((B, M, OUT)), _batch_spec((B, M, 8)),
                 _batch_spec((B, M, 1)), _batch_spec((B, M, 1)))
    fn = pl.pallas_call(body, grid=(B,), in_specs=in_specs,
                        out_specs=out_specs, out_shape=out_shape)
    return fn(x_mid, s_col, s_row, xv_pad, oW, ob)


def kernel(x, x_v, params):
    p = params
    B, N, _ = x.shape
    x2 = x.reshape(B * N, 3)
    xe = _lin_call(_emb_knl, (x2, p['emb_W'], _row(p['emb_b'])),
                   jax.ShapeDtypeStruct((B * N, 128), _F32))
    x = xe.reshape(B, N, 128)
    extras = []
    for i in range(2):
        H, K = _H[i], _K[i]
        B, N, D = x.shape
        dh = D // H
        idx = _knn_idx(x_v, K)
        x2 = x.reshape(B * N, D)
        sd = jax.ShapeDtypeStruct((B * N, D), _F32)
        q, k, v = _lin_call(
            _qkv_knl,
            (x2, p[f'mha{i}_Wq'], _row(p[f'mha{i}_Wqb']),
             p[f'mha{i}_Wk'], _row(p[f'mha{i}_Wkb']),
             p[f'mha{i}_Wv'], _row(p[f'mha{i}_Wvb'])),
            (sd, sd, sd))
        q = q.reshape(B, N, D)
        k = k.reshape(B, N, D)
        v = v.reshape(B, N, D)
        kn2, vn2 = _gather_kv(k, v, idx)
        kn = kn2.reshape(B, N, K, H, dh)
        vn = vn2.reshape(B, N, K, H, dh)
        qh = q.reshape(B, N, H, dh)
        logits = jnp.einsum('bnhd,bnkhd->bnhk', qh, kn) / np.sqrt(dh)
        w = jax.nn.softmax(logits, axis=-1)
        o = jnp.einsum('bnhk,bnkhd->bnhd', w, vn).reshape(B * N, D)
        xa = _lin_call(_res_knl,
                       (x2, o, p[f'mha{i}_Wo'], _row(p[f'mha{i}_Wob'])), sd)
        FFD = p[f'ff{i}_W1'].shape[1]
        PH = p[f'pool{i}_W1'].shape[1]
        xm, h = _lin_call(
            _ff_knl,
            (xa, p[f'ff{i}_W1'], _row(p[f'ff{i}_b1']),
             p[f'ff{i}_W2'], _row(p[f'ff{i}_b2']),
             p[f'pool{i}_W1'], _row(p[f'pool{i}_b1'])),
            (sd, jax.ShapeDtypeStruct((B * N, PH), _F32)))
        s = (h @ p[f'pool{i}_W2'] + p[f'pool{i}_b2'])[..., 0].reshape(B, N)
        M = int(N * _PF[i])
        OUT = p[f'out{i}_W'].shape[1]
        xv_pad = jnp.pad(x_v, ((0, 0), (0, 0), (0, 5)))
        xn, xvn, vals, idxs = _run_pool(
            xm.reshape(B, N, D), s, xv_pad, p[f'out{i}_W'],
            _row(p[f'out{i}_b']), M=M, OUT=OUT)
        x = xn
        x_v = xvn[..., :3]
        extras.append((vals[..., 0], idxs[..., 0]))
    return (x, x_v, extras[0][0], extras[0][1], extras[1][0], extras[1][1])


# speedup vs baseline: 2.0878x; 1.0004x over previous
import functools

import jax
import jax.numpy as jnp
import numpy as np
from jax import lax
from jax.experimental import pallas as pl

_H = (8, 16)
_K = (32, 32)
_PF = (0.25, 0.25)
_BF = jnp.bfloat16
_F32 = jnp.float32
_HI = lax.Precision.HIGHEST


def _bdot(a, b):
    return jnp.dot(a.astype(_BF), b.astype(_BF), preferred_element_type=_F32)


def _qkv_knl(x_ref, wq, bq, wk, bk, wv, bv, q_ref, k_ref, v_ref):
    x = x_ref[...]
    q_ref[...] = _bdot(x, wq[...]) + bq[...]
    k_ref[...] = _bdot(x, wk[...]) + bk[...]
    v_ref[...] = _bdot(x, wv[...]) + bv[...]


def _res_knl(x_ref, o_ref, w, b, out_ref):
    out_ref[...] = x_ref[...] + _bdot(o_ref[...], w[...]) + b[...]


def _ff_knl(x_ref, w1, b1, w2, b2, p1, pb1, xm_ref, h_ref):
    x = x_ref[...]
    ff = jnp.maximum(_bdot(x, w1[...]) + b1[...], 0.0)
    xm = x + _bdot(ff, w2[...]) + b2[...]
    xm_ref[...] = xm
    h_ref[...] = jnp.tanh(_bdot(xm, p1[...]) + pb1[...])


def _emb_knl(x_ref, w, b, out_ref):
    out_ref[...] = _bdot(x_ref[...], w[...]) + b[...]


def _pool_knl(x_ref, sc_ref, sr_ref, xv_ref, oW_ref, ob_ref,
              xo_ref, xvo_ref, vals_ref, idx_ref, *, N, M, RC):
    s_col = sc_ref[0]
    s_row = sr_ref[0]
    parts = []
    for c in range(N // RC):
        sn = s_row[:, c * RC:(c + 1) * RC]
        gt = (s_col > sn).astype(_F32)
        eq = s_col == sn
        jj = lax.broadcasted_iota(jnp.int32, (N, RC), 0)
        nn = lax.broadcasted_iota(jnp.int32, (N, RC), 1) + c * RC
        tie = jnp.where(eq, (jj < nn).astype(_F32), 0.0)
        parts.append(jnp.sum(gt + tie, axis=0, keepdims=True))
    rank = jnp.concatenate(parts, axis=1)
    om = lax.broadcasted_iota(jnp.int32, (M, 1), 0).astype(_F32)
    P = (rank == om).astype(_F32)
    x_pool = jnp.dot(P, x_ref[0], preferred_element_type=_F32, precision=_HI)
    xv_pool = jnp.dot(P, xv_ref[0], preferred_element_type=_F32, precision=_HI)
    vals = jnp.dot(P, s_col, preferred_element_type=_F32, precision=_HI)
    ii = lax.broadcasted_iota(jnp.int32, (N, 1), 0).astype(_F32)
    idxf = jnp.dot(P, ii, preferred_element_type=_F32, precision=_HI)
    gate = jax.nn.sigmoid(vals)
    xo_ref[0] = _bdot(x_pool * gate, oW_ref[...]) + ob_ref[...]
    xvo_ref[0] = xv_pool
    vals_ref[0] = vals
    idx_ref[0] = jnp.round(idxf).astype(jnp.int32)


def _gat_knl(idx_ref, p0_ref, p1_ref, p2_ref, kn_ref, vn_ref, *, CH, N, D):
    idx_col = idx_ref[0]
    iota = lax.broadcasted_iota(jnp.int32, (CH, N), 1)
    G = (idx_col == iota).astype(_BF)
    a = jnp.dot(G, p0_ref[0], preferred_element_type=_F32)
    b = jnp.dot(G, p1_ref[0], preferred_element_type=_F32)
    c = jnp.dot(G, p2_ref[0], preferred_element_type=_F32)
    kv = (a + b) + c
    kn_ref[0] = kv[:, :D]
    vn_ref[0] = kv[:, D:]


def _bf3_split(a):
    bits = lax.bitcast_convert_type(a, jnp.uint32)
    p0 = lax.bitcast_convert_type(bits & jnp.uint32(0xFFFF0000), jnp.float32)
    r1 = a - p0
    bits1 = lax.bitcast_convert_type(r1, jnp.uint32)
    p1 = lax.bitcast_convert_type(bits1 & jnp.uint32(0xFFFF0000), jnp.float32)
    p2 = r1 - p1
    return p0.astype(_BF), p1.astype(_BF), p2.astype(_BF)


def _gather_kv(k, v, idx):
    B, N, D = k.shape
    K = idx.shape[-1]
    NK = N * K
    CH = 2048 if NK % 2048 == 0 else 512
    NC = NK // CH
    idx3 = idx.reshape(B * NC, CH, 1)
    kv = jnp.concatenate([k, v], axis=-1)
    ps = [p for p in _bf3_split(kv)]
    body = functools.partial(_gat_knl, CH=CH, N=N, D=D)
    in_specs = [pl.BlockSpec((1, CH, 1), lambda b, c: (b * NC + c, 0, 0))]
    in_specs += [pl.BlockSpec((1, N, 2 * D), lambda b, c: (b, 0, 0))] * 3
    out_specs = (pl.BlockSpec((1, CH, D), lambda b, c: (b, c, 0)),) * 2
    out_shape = (jax.ShapeDtypeStruct((B, NK, D), _F32),) * 2
    fn = pl.pallas_call(body, grid=(B, NC), in_specs=in_specs,
                        out_specs=out_specs, out_shape=out_shape)
    kn, vn = fn(idx3, *ps)
    return kn, vn


def _lin_call(knl, args, out_shapes):
    return pl.pallas_call(knl, out_shape=out_shapes)(*args)


def _row(a):
    return a.reshape(1, -1)


def _knn_idx(xv, K):
    n2 = jnp.sum(xv * xv, axis=-1)
    d2 = n2[:, :, None] + n2[:, None, :] - 2.0 * jnp.einsum('bnd,bmd->bnm', xv, xv)
    _, idx = jax.lax.top_k(-d2, K)
    return idx


def _batch_spec(shape):
    nd = len(shape)
    return pl.BlockSpec((1,) + shape[1:], lambda b: (b,) + (0,) * (nd - 1))


def _full_spec(shape):
    nd = len(shape)
    return pl.BlockSpec(shape, lambda b: (0,) * nd)


def _run_pool(x_mid, s, xv_pad, oW, ob, *, M, OUT):
    B, N, D = x_mid.shape
    s_col = s[..., None]
    s_row = s[:, None, :]
    body = functools.partial(_pool_knl, N=N, M=M, RC=512)
    in_specs = [_batch_spec(x_mid.shape), _batch_spec(s_col.shape),
                _batch_spec(s_row.shape), _batch_spec(xv_pad.shape),
                _full_spec(oW.shape), _full_spec(ob.shape)]
    out_shape = (jax.ShapeDtypeStruct((B, M, OUT), _F32),
                 jax.ShapeDtypeStruct((B, M, 8), _F32),
                 jax.ShapeDtypeStruct((B, M, 1), _F32),
                 jax.ShapeDtypeStruct((B, M, 1), jnp.int32))
    out_specs = (_batch_spec((B, M, OUT)), _batch_spec((B, M, 8)),
                 _batch_spec((B, M, 1)), _batch_spec((B, M, 1)))
    fn = pl.pallas_call(body, grid=(B,), in_specs=in_specs,
                        out_specs=out_specs, out_shape=out_shape)
    return fn(x_mid, s_col, s_row, xv_pad, oW, ob)


def kernel(x, x_v, params):
    p = params
    B, N, _ = x.shape
    x2 = x.reshape(B * N, 3)
    xe = _lin_call(_emb_knl, (x2, p['emb_W'], _row(p['emb_b'])),
                   jax.ShapeDtypeStruct((B * N, 128), _F32))
    x = xe.reshape(B, N, 128)
    extras = []
    for i in range(2):
        H, K = _H[i], _K[i]
        B, N, D = x.shape
        dh = D // H
        idx = _knn_idx(x_v, K)
        x2 = x.reshape(B * N, D)
        sd = jax.ShapeDtypeStruct((B * N, D), _F32)
        q, k, v = _lin_call(
            _qkv_knl,
            (x2, p[f'mha{i}_Wq'], _row(p[f'mha{i}_Wqb']),
             p[f'mha{i}_Wk'], _row(p[f'mha{i}_Wkb']),
             p[f'mha{i}_Wv'], _row(p[f'mha{i}_Wvb'])),
            (sd, sd, sd))
        q = q.reshape(B, N, D)
        k = k.reshape(B, N, D)
        v = v.reshape(B, N, D)
        kn2, vn2 = _gather_kv(k, v, idx)
        kn = kn2.reshape(B, N, K, H, dh)
        vn = vn2.reshape(B, N, K, H, dh)
        qh = q.reshape(B, N, H, dh)
        logits = jnp.einsum('bnhd,bnkhd->bnhk', qh, kn) / np.sqrt(dh)
        w = jax.nn.softmax(logits, axis=-1)
        o = jnp.einsum('bnhk,bnkhd->bnhd', w, vn).reshape(B * N, D)
        xa = _lin_call(_res_knl,
                       (x2, o, p[f'mha{i}_Wo'], _row(p[f'mha{i}_Wob'])), sd)
        FFD = p[f'ff{i}_W1'].shape[1]
        PH = p[f'pool{i}_W1'].shape[1]
        xm, h = _lin_call(
            _ff_knl,
            (xa, p[f'ff{i}_W1'], _row(p[f'ff{i}_b1']),
             p[f'ff{i}_W2'], _row(p[f'ff{i}_b2']),
             p[f'pool{i}_W1'], _row(p[f'pool{i}_b1'])),
            (sd, jax.ShapeDtypeStruct((B * N, PH), _F32)))
        s = (h @ p[f'pool{i}_W2'] + p[f'pool{i}_b2'])[..., 0].reshape(B, N)
        M = int(N * _PF[i])
        OUT = p[f'out{i}_W'].shape[1]
        xv_pad = jnp.pad(x_v, ((0, 0), (0, 0), (0, 5)))
        xn, xvn, vals, idxs = _run_pool(
            xm.reshape(B, N, D), s, xv_pad, p[f'out{i}_W'],
            _row(p[f'out{i}_b']), M=M, OUT=OUT)
        x = xn
        x_v = xvn[..., :3]
        extras.append((vals[..., 0], idxs[..., 0]))
    return (x, x_v, extras[0][0], extras[0][1], extras[1][0], extras[1][1])
